# baseline (speedup 1.0000x reference)
import functools

import jax
import jax.numpy as jnp
from jax import lax
from jax.experimental import pallas as pl
from jax.experimental.pallas import tpu as pltpu
from jax.experimental.pallas import tpu_sc as plsc

NC = 2
NS = 16
NW = NC * NS
K = 128
N_PAD = 10240
SPLIT1 = (80, 78)
SPLIT2 = (80, 78)


def _a_body(x, win, bin_, ws1, bs1, wn1, bn1, m1, m1b, s1):
    h = jnp.maximum(
        jnp.dot(x[...], win[...], preferred_element_type=jnp.float32)
        + bin_[...], 0.0)
    mm = jnp.dot(h, wn1[...], preferred_element_type=jnp.float32)
    m1[...] = mm
    m1b[...] = mm
    s1[...] = (jnp.dot(h, ws1[...], preferred_element_type=jnp.float32)
               + bs1[...] + bn1[...])


def _c_body(s1, p0, p1, d0, d1, ws2, bs2, wn2, bn2, m2, m2b, s2):
    deg = jnp.maximum(d0[...] + d1[...], 1e-12)
    h1 = jnp.maximum(s1[...] + (p0[...] + p1[...]) / deg, 0.0)
    mm = jnp.dot(h1, wn2[...], preferred_element_type=jnp.float32)
    m2[...] = mm
    m2b[...] = mm
    s2[...] = (jnp.dot(h1, ws2[...], preferred_element_type=jnp.float32)
               + bs2[...] + bn2[...])


def _e_body(s2, q0, q1, d0, d1, out):
    deg = jnp.maximum(d0[...] + d1[...], 1e-12)
    out[...] = s2[...] + (q0[...] + q1[...]) / deg


def _full(shape):
    return pl.BlockSpec(shape, lambda i: (0, 0))


def _rows(br, d):
    return pl.BlockSpec((br, d), lambda i: (i, 0))


@functools.cache
def _make_segsum(steps0, steps1, d, with_deg, tc_tiling):
    assert steps0 % 2 == 0 and steps1 % 2 == 0 and steps0 >= steps1
    per_tile = N_PAD // NS
    mesh = plsc.VectorSubcoreMesh(core_axis_name="c", subcore_axis_name="s",
                                  num_cores=NC, num_subcores=NS)

    out_type = [jax.ShapeDtypeStruct((N_PAD, d), jnp.float32),
                jax.ShapeDtypeStruct((N_PAD, d), jnp.float32)]
    scratch = [pltpu.VMEM((steps0, K), jnp.int32),
               pltpu.VMEM((K,), jnp.int32),
               pltpu.VMEM((K,), jnp.int32),
               pltpu.VMEM((K,), jnp.int32),
               pltpu.VMEM((K,), jnp.int32),
               pltpu.VMEM((K, d), jnp.float32),
               pltpu.VMEM((K, d), jnp.float32),
               pltpu.VMEM_SHARED((N_PAD, d), jnp.float32),
               pltpu.SemaphoreType.DMA,
               pltpu.SemaphoreType.DMA]
    if with_deg:
        out_type.append(jax.ShapeDtypeStruct((NC * N_PAD,), jnp.float32))
        scratch.append(pltpu.VMEM((K,), jnp.float32))
        scratch.append(pltpu.VMEM_SHARED((N_PAD,), jnp.float32))

    def make_body():
        def body(*refs):
            if with_deg:
                (ma_hbm, mb_hbm, pk_hbm, zrow_hbm, zdeg_hbm, out0_hbm,
                 out1_hbm, deg_hbm, pkv, row0, row1, col0, col1, bufa, bufb,
                 agg_sh, sema, semb, onesv, deg_sh) = refs
            else:
                (ma_hbm, mb_hbm, pk_hbm, zrow_hbm, out0_hbm, out1_hbm,
                 pkv, row0, row1, col0, col1, bufa, bufb, agg_sh,
                 sema, semb) = refs
            c = lax.axis_index("c")
            s = lax.axis_index("s")
            wid = c * NS + s
            steps = jnp.where(c == 0, steps0, steps1)
            rows = (row0, row1)
            cols = (col0, col1)
            bufs = (bufa, bufb)
            sems = (sema, semb)

            pltpu.sync_copy(pk_hbm.at[wid], pkv)
            pltpu.sync_copy(zrow_hbm, agg_sh.at[pl.ds(s * per_tile, per_tile)])
            if with_deg:
                pltpu.sync_copy(zdeg_hbm,
                                deg_sh.at[pl.ds(s * per_tile, per_tile)])
                for j in range(K // 16):
                    onesv[pl.ds(j * 16, 16)] = jnp.ones((16,), jnp.float32)
            plsc.subcore_barrier()

            def run(m_hbm):
                def start(t, b):
                    for j in range(K // 16):
                        v = pkv[t, pl.ds(j * 16, 16)]
                        rows[b][pl.ds(j * 16, 16)] = v & 0xFFFF
                        cols[b][pl.ds(j * 16, 16)] = (
                            lax.shift_right_logical(v, 16))
                    pltpu.async_copy(m_hbm.at[cols[b]], bufs[b], sems[b])

                def finish(t, b):
                    pltpu.make_async_copy(m_hbm.at[cols[b]], bufs[b],
                                          sems[b]).wait()
                    pltpu.sync_copy(bufs[b], agg_sh.at[rows[b]], add=True)
                    if with_deg:
                        pltpu.sync_copy(onesv, deg_sh.at[rows[b]], add=True)

                start(0, 0)
                start(1, 1)

                def pair(i, carry):
                    t0 = 2 * i
                    finish(t0, 0)

                    @pl.when(t0 + 2 < steps)
                    def _():
                        start(t0 + 2, 0)
                    finish(t0 + 1, 1)

                    @pl.when(t0 + 3 < steps)
                    def _():
                        start(t0 + 3, 1)
                    return carry

                lax.fori_loop(0, steps // 2, pair, 0)

            @pl.when(c == 0)
            def _():
                run(ma_hbm)

            @pl.when(c == 1)
            def _():
                run(mb_hbm)
            plsc.subcore_barrier()

            sl = pl.ds(s * per_tile, per_tile)

            @pl.when(c == 0)
            def _():
                pltpu.sync_copy(agg_sh.at[sl], out0_hbm.at[sl])

            @pl.when(c == 1)
            def _():
                pltpu.sync_copy(agg_sh.at[sl], out1_hbm.at[sl])
            if with_deg:
                pltpu.sync_copy(deg_sh.at[sl],
                                deg_hbm.at[pl.ds(c * N_PAD + s * per_tile,
                                                 per_tile)])
        return body

    params = pltpu.CompilerParams(use_tc_tiling_on_sc=tc_tiling)
    return pl.kernel(make_body(), mesh=mesh, out_type=tuple(out_type),
                     scratch_types=tuple(scratch), compiler_params=params)


def kernel(x, edge_index, W_in, b_in, W_self1, b_self1, W_neigh1, b_neigh1,
           W_self2, b_self2, W_neigh2, b_neigh2, enable_rewire=False):
    n, d_in = x.shape
    d_h = W_in.shape[1]
    d_out = W_self2.shape[1]
    e_total = edge_index.shape[1]
    br = 1000
    grid = (n // br,)

    chunk = NW * K
    n_chunks = -(-e_total // chunk) * NW
    e_pad = n_chunks * K
    pad = e_pad - e_total
    pad_rows = n + jnp.arange(pad, dtype=jnp.int32) % (N_PAD - n)
    row_p = jnp.concatenate([edge_index[0], pad_rows])
    col_p = jnp.concatenate([edge_index[1],
                             jnp.zeros((pad,), jnp.int32)])
    chunks = (row_p | (col_p << 16)).reshape(n_chunks, K)

    def split_chunks(steps0, steps1):
        a = chunks[:NS * steps0].reshape(NS, steps0, K)
        b = chunks[NS * steps0:].reshape(NS, steps1, K)
        b = jnp.pad(b, ((0, 0), (0, steps0 - steps1), (0, 0)))
        return jnp.concatenate([a, b], axis=0)

    st0_1, st1_1 = SPLIT1
    st0_2, st1_2 = SPLIT2
    pk1 = split_chunks(st0_1, st1_1)
    pk2 = split_chunks(st0_2, st1_2)

    zrow_h = jnp.zeros((N_PAD // NS, d_h), jnp.float32)
    zrow_o = jnp.zeros((N_PAD // NS, d_out), jnp.float32)
    zdeg = jnp.zeros((N_PAD // NS,), jnp.float32)

    b_in2 = b_in.reshape(1, d_h)
    bs1 = b_self1.reshape(1, d_h)
    bn1 = b_neigh1.reshape(1, d_h)
    bs2 = b_self2.reshape(1, d_out)
    bn2 = b_neigh2.reshape(1, d_out)

    m1, m1b, s1 = pl.pallas_call(
        _a_body,
        grid=grid,
        in_specs=[_rows(br, d_in), _full((d_in, d_h)), _full((1, d_h)),
                  _full((d_h, d_h)), _full((1, d_h)),
                  _full((d_h, d_h)), _full((1, d_h))],
        out_specs=[_rows(br, d_h), _rows(br, d_h), _rows(br, d_h)],
        out_shape=[jax.ShapeDtypeStruct((n, d_h), jnp.float32),
                   jax.ShapeDtypeStruct((n, d_h), jnp.float32),
                   jax.ShapeDtypeStruct((n, d_h), jnp.float32)],
    )(x, W_in, b_in2, W_self1, bs1, W_neigh1, bn1)

    segsum1 = _make_segsum(st0_1, st1_1, d_h, True, True)
    p0, p1, deg = segsum1(m1, m1b, pk1, zrow_h, zdeg)
    d0 = deg[:n].reshape(n, 1)
    d1 = deg[N_PAD:N_PAD + n].reshape(n, 1)

    m2, m2b, s2 = pl.pallas_call(
        _c_body,
        grid=grid,
        in_specs=[_rows(br, d_h), _rows(br, d_h), _rows(br, d_h),
                  _rows(br, 1), _rows(br, 1),
                  _full((d_h, d_out)), _full((1, d_out)),
                  _full((d_h, d_out)), _full((1, d_out))],
        out_specs=[_rows(br, d_out), _rows(br, d_out), _rows(br, d_out)],
        out_shape=[jax.ShapeDtypeStruct((n, d_out), jnp.float32),
                   jax.ShapeDtypeStruct((n, d_out), jnp.float32),
                   jax.ShapeDtypeStruct((n, d_out), jnp.float32)],
    )(s1, p0, p1, d0, d1, W_self2, bs2, W_neigh2, bn2)

    segsum2 = _make_segsum(st0_2, st1_2, d_out, False, False)
    q0, q1 = segsum2(m2, m2b, pk2, zrow_o)

    (logits,) = pl.pallas_call(
        _e_body,
        grid=grid,
        in_specs=[_rows(br, d_out), _rows(br, d_out), _rows(br, d_out),
                  _rows(br, 1), _rows(br, 1)],
        out_specs=[_rows(br, d_out)],
        out_shape=[jax.ShapeDtypeStruct((n, d_out), jnp.float32)],
    )(s2, q0, q1, d0, d1)

    return logits


# speedup vs baseline: 1.3075x; 1.3075x over previous
import functools

import jax
import jax.numpy as jnp
from jax import lax
from jax.experimental import pallas as pl
from jax.experimental.pallas import tpu as pltpu
from jax.experimental.pallas import tpu_sc as plsc

NC = 2
NS = 16
NW = NC * NS
K = 128
N_PAD = 10240


def _a_body(x, win, bin_, ws1, bs1, wn1, bn1, m1, m1b, s1):
    h = jnp.maximum(
        jnp.dot(x[...], win[...], preferred_element_type=jnp.float32)
        + bin_[...], 0.0)
    mm = jnp.dot(h, wn1[...], preferred_element_type=jnp.float32)
    m1[...] = mm
    m1b[...] = mm
    s1[...] = (jnp.dot(h, ws1[...], preferred_element_type=jnp.float32)
               + bs1[...] + bn1[...])


def _c_body(s1, p0, p1, d0, d1, ws2, bs2, wn2, bn2, m2, m2b, s2):
    deg = jnp.maximum(d0[...] + d1[...], 1e-12)
    h1 = jnp.maximum(s1[...] + (p0[...] + p1[...]) / deg, 0.0)
    mm = jnp.dot(h1, wn2[...], preferred_element_type=jnp.float32)
    m2[...] = mm
    m2b[...] = mm
    s2[...] = (jnp.dot(h1, ws2[...], preferred_element_type=jnp.float32)
               + bs2[...] + bn2[...])


def _e_body(s2, q0, q1, d0, d1, out):
    deg = jnp.maximum(d0[...] + d1[...], 1e-12)
    out[...] = s2[...] + (q0[...] + q1[...]) / deg


def _full(shape):
    return pl.BlockSpec(shape, lambda i: (0, 0))


def _rows(br, d):
    return pl.BlockSpec((br, d), lambda i: (i, 0))


@functools.cache
def _make_segsum(e_pad, d, with_deg, tc_tiling):
    steps = e_pad // (NW * K)
    per_tile = N_PAD // NS
    mesh = plsc.VectorSubcoreMesh(core_axis_name="c", subcore_axis_name="s",
                                  num_cores=NC, num_subcores=NS)

    out_type = [jax.ShapeDtypeStruct((N_PAD, d), jnp.float32),
                jax.ShapeDtypeStruct((N_PAD, d), jnp.float32)]
    scratch = [pltpu.VMEM((steps, K), jnp.int32),
               pltpu.VMEM((K,), jnp.int32),
               pltpu.VMEM((K,), jnp.int32),
               pltpu.VMEM((K,), jnp.int32),
               pltpu.VMEM((K,), jnp.int32),
               pltpu.VMEM((K, d), jnp.float32),
               pltpu.VMEM((K, d), jnp.float32),
               pltpu.VMEM_SHARED((N_PAD, d), jnp.float32),
               pltpu.SemaphoreType.DMA,
               pltpu.SemaphoreType.DMA]
    if with_deg:
        out_type.append(jax.ShapeDtypeStruct((NC * N_PAD,), jnp.float32))
        scratch.append(pltpu.VMEM((K,), jnp.float32))
        scratch.append(pltpu.VMEM_SHARED((N_PAD,), jnp.float32))

    def body(*refs):
        if with_deg:
            (ma_hbm, mb_hbm, pk_hbm, zrow_hbm, zdeg_hbm, out0_hbm,
             out1_hbm, deg_hbm, pkv, row0, row1, col0, col1, bufa, bufb,
             agg_sh, sema, semb, onesv, deg_sh) = refs
        else:
            (ma_hbm, mb_hbm, pk_hbm, zrow_hbm, out0_hbm, out1_hbm,
             pkv, row0, row1, col0, col1, bufa, bufb, agg_sh,
             sema, semb) = refs
        c = lax.axis_index("c")
        s = lax.axis_index("s")
        wid = c * NS + s
        rows = (row0, row1)
        cols = (col0, col1)
        bufs = (bufa, bufb)
        sems = (sema, semb)

        pltpu.sync_copy(pk_hbm.at[wid], pkv)
        pltpu.sync_copy(zrow_hbm, agg_sh.at[pl.ds(s * per_tile, per_tile)])
        if with_deg:
            pltpu.sync_copy(zdeg_hbm,
                            deg_sh.at[pl.ds(s * per_tile, per_tile)])
            for j in range(K // 16):
                onesv[pl.ds(j * 16, 16)] = jnp.ones((16,), jnp.float32)
        plsc.subcore_barrier()

        def run(m_hbm):
            def start(t, b):
                for j in range(K // 16):
                    v = pkv[t, pl.ds(j * 16, 16)]
                    rows[b][pl.ds(j * 16, 16)] = v & 0xFFFF
                    cols[b][pl.ds(j * 16, 16)] = (
                        lax.shift_right_logical(v, 16))
                pltpu.async_copy(m_hbm.at[cols[b]], bufs[b], sems[b])

            def finish(t, b):
                pltpu.make_async_copy(m_hbm.at[cols[b]], bufs[b],
                                      sems[b]).wait()
                pltpu.sync_copy(bufs[b], agg_sh.at[rows[b]], add=True)
                if with_deg:
                    pltpu.sync_copy(onesv, deg_sh.at[rows[b]], add=True)

            start(0, 0)
            start(1, 1)

            def pair(i, carry):
                t0 = 2 * i
                finish(t0, 0)

                @pl.when(t0 + 2 < steps)
                def _():
                    start(t0 + 2, 0)
                finish(t0 + 1, 1)

                @pl.when(t0 + 3 < steps)
                def _():
                    start(t0 + 3, 1)
                return carry

            lax.fori_loop(0, steps // 2, pair, 0)
            if steps % 2 == 1:
                finish(steps - 1, 0)

        @pl.when(c == 0)
        def _():
            run(ma_hbm)

        @pl.when(c == 1)
        def _():
            run(mb_hbm)
        plsc.subcore_barrier()

        sl = pl.ds(s * per_tile, per_tile)

        @pl.when(c == 0)
        def _():
            pltpu.sync_copy(agg_sh.at[sl], out0_hbm.at[sl])

        @pl.when(c == 1)
        def _():
            pltpu.sync_copy(agg_sh.at[sl], out1_hbm.at[sl])
        if with_deg:
            pltpu.sync_copy(deg_sh.at[sl],
                            deg_hbm.at[pl.ds(c * N_PAD + s * per_tile,
                                             per_tile)])

    params = pltpu.CompilerParams(use_tc_tiling_on_sc=tc_tiling)
    return pl.kernel(body, mesh=mesh, out_type=tuple(out_type),
                     scratch_types=tuple(scratch), compiler_params=params)


def kernel(x, edge_index, W_in, b_in, W_self1, b_self1, W_neigh1, b_neigh1,
           W_self2, b_self2, W_neigh2, b_neigh2, enable_rewire=False):
    n, d_in = x.shape
    d_h = W_in.shape[1]
    d_out = W_self2.shape[1]
    e_total = edge_index.shape[1]
    br = 1000
    grid = (n // br,)

    chunk = NW * K
    steps = -(-e_total // chunk)
    e_pad = steps * chunk
    pad = e_pad - e_total
    pad_rows = n + jnp.arange(pad, dtype=jnp.int32) % (N_PAD - n)
    row_p = jnp.concatenate([edge_index[0], pad_rows])
    col_p = jnp.concatenate([edge_index[1],
                             jnp.zeros((pad,), jnp.int32)])
    pk = (row_p | (col_p << 16)).reshape(steps, NW, K).transpose(1, 0, 2)

    zrow_h = jnp.zeros((N_PAD // NS, d_h), jnp.float32)
    zrow_o = jnp.zeros((N_PAD // NS, d_out), jnp.float32)
    zdeg = jnp.zeros((N_PAD // NS,), jnp.float32)

    b_in2 = b_in.reshape(1, d_h)
    bs1 = b_self1.reshape(1, d_h)
    bn1 = b_neigh1.reshape(1, d_h)
    bs2 = b_self2.reshape(1, d_out)
    bn2 = b_neigh2.reshape(1, d_out)

    m1, m1b, s1 = pl.pallas_call(
        _a_body,
        grid=grid,
        in_specs=[_rows(br, d_in), _full((d_in, d_h)), _full((1, d_h)),
                  _full((d_h, d_h)), _full((1, d_h)),
                  _full((d_h, d_h)), _full((1, d_h))],
        out_specs=[_rows(br, d_h), _rows(br, d_h), _rows(br, d_h)],
        out_shape=[jax.ShapeDtypeStruct((n, d_h), jnp.float32),
                   jax.ShapeDtypeStruct((n, d_h), jnp.float32),
                   jax.ShapeDtypeStruct((n, d_h), jnp.float32)],
    )(x, W_in, b_in2, W_self1, bs1, W_neigh1, bn1)

    segsum1 = _make_segsum(e_pad, d_h, True, True)
    p0, p1, deg = segsum1(m1, m1b, pk, zrow_h, zdeg)
    d0 = deg[:n].reshape(n, 1)
    d1 = deg[N_PAD:N_PAD + n].reshape(n, 1)

    m2, m2b, s2 = pl.pallas_call(
        _c_body,
        grid=grid,
        in_specs=[_rows(br, d_h), _rows(br, d_h), _rows(br, d_h),
                  _rows(br, 1), _rows(br, 1),
                  _full((d_h, d_out)), _full((1, d_out)),
                  _full((d_h, d_out)), _full((1, d_out))],
        out_specs=[_rows(br, d_out), _rows(br, d_out), _rows(br, d_out)],
        out_shape=[jax.ShapeDtypeStruct((n, d_out), jnp.float32),
                   jax.ShapeDtypeStruct((n, d_out), jnp.float32),
                   jax.ShapeDtypeStruct((n, d_out), jnp.float32)],
    )(s1, p0, p1, d0, d1, W_self2, bs2, W_neigh2, bn2)

    segsum2 = _make_segsum(e_pad, d_out, False, False)
    q0, q1 = segsum2(m2, m2b, pk, zrow_o)

    (logits,) = pl.pallas_call(
        _e_body,
        grid=grid,
        in_specs=[_rows(br, d_out), _rows(br, d_out), _rows(br, d_out),
                  _rows(br, 1), _rows(br, 1)],
        out_specs=[_rows(br, d_out)],
        out_shape=[jax.ShapeDtypeStruct((n, d_out), jnp.float32)],
    )(s2, q0, q1, d0, d1)

    return logits
